# Initial kernel scaffold; baseline (speedup 1.0000x reference)
import jax
import jax.numpy as jnp
from jax.experimental import pallas as pl


def kernel(feats, edge_index, W1, b1, W2, b2):
    raise NotImplementedError("write your pallas kernel here")


# speedup vs baseline: 3.8252x; 3.8252x over previous
import jax
import jax.numpy as jnp
from jax import lax
from jax.experimental import pallas as pl
from jax.experimental.pallas import tpu as pltpu, tpu_sc as plsc

N = 10000
E = 320000
D = 128
H = 256
C = 64
SAMPLE = 2
ORDER = 4
P_DROP_NODE = 0.5

NS = 16
LANES = 16
EPS = E // NS
CH = 80
NCH = EPS // CH
RCH = 80
NRC = N // RCH
RITER = (NRC + NS - 1) // NS
DV = D // LANES


def _rsqrt16(x):
    y = jnp.full((LANES,), 1.0, jnp.float32)
    for k in range(1, 19):
        y = jnp.where(x >= float(1 << k), jnp.float32(2.0 ** (-k / 2.0)), y)
    for _ in range(5):
        y = y * (1.5 - 0.5 * x * y * y)
    return y


def _prop_body(x0_hbm, src2_hbm, dst_hbm,
               xs_hbm, nx1_hbm, nx2_hbm, nx3_hbm, nx4_hbm, nsh_hbm, ndh_hbm,
               acc, idxs, idxd, msg, rowbuf, nbuf, sem):
    c = lax.axis_index("c")
    s = lax.axis_index("s")
    nx_hbm = [nx1_hbm, nx2_hbm, nx3_hbm, nx4_hbm]

    def zero_rowbuf(r, _):
        for j in range(DV):
            rowbuf[r, pl.ds(LANES * j, LANES)] = jnp.zeros((LANES,), jnp.float32)
        return 0

    def srow(r, _c):
        nv = nbuf[r, :]
        for j in range(DV):
            sl = pl.ds(LANES * j, LANES)
            rowbuf[r, sl] = rowbuf[r, sl] * nv
        return 0

    def fill_body(r, _):
        zero_rowbuf(r, 0)
        for j in range(DV):
            msg[r, pl.ds(LANES * j, LANES)] = jnp.ones((LANES,), jnp.float32)
        return 0

    lax.fori_loop(0, RCH, fill_body, 0)

    def zero_body(i, _):
        ch = i * NS + s

        @pl.when(ch < NRC)
        def _():
            pltpu.sync_copy(rowbuf, acc.at[pl.ds(ch * RCH, RCH)])

        return 0

    lax.fori_loop(0, RITER, zero_body, 0)
    plsc.subcore_barrier()

    for which, norm_hbm in ((0, nsh_hbm), (1, ndh_hbm)):
        def deg_body(i, _):
            off = s * EPS + i * CH
            if which == 0:
                pltpu.sync_copy(src2_hbm.at[pl.ds(off, CH)], idxs)
            else:
                pltpu.sync_copy(dst_hbm.at[pl.ds(off, CH)], idxs)
            pltpu.sync_copy(msg, acc.at[idxs], add=True)
            return 0

        lax.fori_loop(0, NCH, deg_body, 0)
        plsc.subcore_barrier()

        def extract_body(i, _):
            ch = i * NS + s

            @pl.when(ch < NRC)
            def _():
                row0 = ch * RCH
                pltpu.sync_copy(acc.at[pl.ds(row0, RCH)], rowbuf)

                def nrow(r, _c):
                    nbuf[r, :] = _rsqrt16(
                        jnp.maximum(rowbuf[r, pl.ds(0, LANES)], 1.0))
                    return 0

                lax.fori_loop(0, RCH, nrow, 0)
                pltpu.sync_copy(nbuf, norm_hbm.at[pl.ds(row0, RCH)])
                lax.fori_loop(0, RCH, zero_rowbuf, 0)
                pltpu.sync_copy(rowbuf, acc.at[pl.ds(row0, RCH)])

            return 0

        lax.fori_loop(0, RITER, extract_body, 0)
        plsc.subcore_barrier()

    def init_body(i, _):
        ch = i * NS + s

        @pl.when(ch < NRC)
        def _():
            row0 = ch * RCH
            grow0 = c * N + row0
            pltpu.sync_copy(nsh_hbm.at[pl.ds(row0, RCH)], nbuf)
            pltpu.sync_copy(x0_hbm.at[pl.ds(grow0, RCH)], rowbuf)
            lax.fori_loop(0, RCH, srow, 0)
            pltpu.sync_copy(rowbuf, xs_hbm.at[pl.ds(grow0, RCH)])

        return 0

    lax.fori_loop(0, RITER, init_body, 0)
    plsc.subcore_barrier()

    for k in range(ORDER):
        last = k == ORDER - 1

        def edge_body(i, _):
            off = s * EPS + i * CH
            pltpu.sync_copy(src2_hbm.at[pl.ds(c * E + off, CH)], idxs)
            pltpu.sync_copy(dst_hbm.at[pl.ds(off, CH)], idxd)
            pltpu.async_copy(xs_hbm.at[idxs], msg, sem).wait()
            pltpu.sync_copy(msg, acc.at[idxd], add=True)
            return 0

        lax.fori_loop(0, NCH, edge_body, 0)
        plsc.subcore_barrier()

        def epi_body(i, _):
            ch = i * NS + s

            @pl.when(ch < NRC)
            def _():
                row0 = ch * RCH
                grow0 = c * N + row0
                pltpu.sync_copy(acc.at[pl.ds(row0, RCH)], rowbuf)
                pltpu.sync_copy(ndh_hbm.at[pl.ds(row0, RCH)], nbuf)
                lax.fori_loop(0, RCH, srow, 0)
                pltpu.sync_copy(rowbuf, nx_hbm[k].at[pl.ds(grow0, RCH)])
                if not last:
                    pltpu.sync_copy(nsh_hbm.at[pl.ds(row0, RCH)], nbuf)
                    lax.fori_loop(0, RCH, srow, 0)
                    pltpu.sync_copy(rowbuf, xs_hbm.at[pl.ds(grow0, RCH)])
                    lax.fori_loop(0, RCH, zero_rowbuf, 0)
                    pltpu.sync_copy(rowbuf, acc.at[pl.ds(row0, RCH)])

            return 0

        lax.fori_loop(0, RITER, epi_body, 0)
        if not last:
            plsc.subcore_barrier()


_propagate = pl.kernel(
    _prop_body,
    out_type=[jax.ShapeDtypeStruct((SAMPLE * N, D), jnp.float32)
              for _ in range(ORDER + 1)]
             + [jax.ShapeDtypeStruct((N, LANES), jnp.float32)
                for _ in range(2)],
    mesh=plsc.VectorSubcoreMesh(core_axis_name="c", subcore_axis_name="s"),
    scratch_types=[
        pltpu.VMEM_SHARED((N, D), jnp.float32),
        pltpu.VMEM((CH,), jnp.int32),
        pltpu.VMEM((CH,), jnp.int32),
        pltpu.VMEM((CH, D), jnp.float32),
        pltpu.VMEM((RCH, D), jnp.float32),
        pltpu.VMEM((RCH, LANES), jnp.float32),
        pltpu.SemaphoreType.DMA,
    ],
)


def _mlp_body(x0_ref, x1_ref, x2_ref, x3_ref, x4_ref,
              w1_ref, b1_ref, w2_ref, b2_ref, out_ref):
    y = (x0_ref[...] + x1_ref[...] + x2_ref[...] + x3_ref[...] + x4_ref[...])
    y = y * (1.0 / (ORDER + 1))
    h = jnp.dot(y, w1_ref[...], preferred_element_type=jnp.float32) + b1_ref[...]
    h = jnp.maximum(h, 0.0)
    o = jnp.dot(h, w2_ref[...], preferred_element_type=jnp.float32) + b2_ref[...]
    m = jnp.max(o, axis=-1, keepdims=True)
    lse = jnp.log(jnp.sum(jnp.exp(o - m), axis=-1, keepdims=True)) + m
    out_ref[...] = o - lse


_MLP_ROWS = 2000

_mlp = pl.pallas_call(
    _mlp_body,
    grid=(SAMPLE * N // _MLP_ROWS,),
    in_specs=[
        pl.BlockSpec((_MLP_ROWS, D), lambda i: (i, 0)),
        pl.BlockSpec((_MLP_ROWS, D), lambda i: (i, 0)),
        pl.BlockSpec((_MLP_ROWS, D), lambda i: (i, 0)),
        pl.BlockSpec((_MLP_ROWS, D), lambda i: (i, 0)),
        pl.BlockSpec((_MLP_ROWS, D), lambda i: (i, 0)),
        pl.BlockSpec((D, H), lambda i: (0, 0)),
        pl.BlockSpec((1, H), lambda i: (0, 0)),
        pl.BlockSpec((H, C), lambda i: (0, 0)),
        pl.BlockSpec((1, C), lambda i: (0, 0)),
    ],
    out_specs=pl.BlockSpec((_MLP_ROWS, C), lambda i: (i, 0)),
    out_shape=jax.ShapeDtypeStruct((SAMPLE * N, C), jnp.float32),
)


@jax.jit
def kernel(feats, edge_index, W1, b1, W2, b2):
    src = edge_index[0]
    dst = edge_index[1]

    masks = []
    for sm in range(SAMPLE):
        mk = jax.random.fold_in(jax.random.key(1), sm)
        masks.append(
            jax.random.bernoulli(mk, 1.0 - P_DROP_NODE, (N, 1)).astype(jnp.float32)
        )
    x0 = jnp.concatenate([feats * m for m in masks], axis=0)

    src2 = jnp.concatenate([src, src + N], axis=0)

    _, nx1, nx2, nx3, nx4, _, _ = _propagate(x0, src2, dst)
    logits = _mlp(x0, nx1, nx2, nx3, nx4,
                  W1, b1.reshape(1, H), W2, b2.reshape(1, C))
    return logits.reshape(SAMPLE, N, C)


# speedup vs baseline: 5.6956x; 1.4890x over previous
import jax
import jax.numpy as jnp
from jax import lax
from jax.experimental import pallas as pl
from jax.experimental.pallas import tpu as pltpu, tpu_sc as plsc

N = 10000
E = 320000
D = 128
H = 256
C = 64
SAMPLE = 2
ORDER = 4
P_DROP_NODE = 0.5

NS = 16
LANES = 16
EPS = E // NS
CH = 80
NCH = EPS // CH
RCH = 80
NRC = N // RCH
RITER = (NRC + NS - 1) // NS
DV = D // LANES


def _rsqrt16(x):
    y = jnp.full((LANES,), 1.0, jnp.float32)
    for k in range(1, 19):
        y = jnp.where(x >= float(1 << k), jnp.float32(2.0 ** (-k / 2.0)), y)
    for _ in range(5):
        y = y * (1.5 - 0.5 * x * y * y)
    return y


def _prop_body(x0_hbm, src2_hbm, dst_hbm,
               xs_hbm, nx1_hbm, nx2_hbm, nx3_hbm, nx4_hbm, nsh_hbm, ndh_hbm,
               acc, idxs, idxd, idxs2, idxd2, msg, msg2, rowbuf, nbuf, sem, sem2):
    c = lax.axis_index("c")
    s = lax.axis_index("s")
    nx_hbm = [nx1_hbm, nx2_hbm, nx3_hbm, nx4_hbm]

    def zero_rowbuf(r, _):
        for j in range(DV):
            rowbuf[r, pl.ds(LANES * j, LANES)] = jnp.zeros((LANES,), jnp.float32)
        return 0

    def srow(r, _c):
        nv = nbuf[r, :]
        for j in range(DV):
            sl = pl.ds(LANES * j, LANES)
            rowbuf[r, sl] = rowbuf[r, sl] * nv
        return 0

    def fill_body(r, _):
        zero_rowbuf(r, 0)
        for j in range(DV):
            msg[r, pl.ds(LANES * j, LANES)] = jnp.ones((LANES,), jnp.float32)
        return 0

    lax.fori_loop(0, RCH, fill_body, 0)

    def zero_body(i, _):
        ch = i * NS + s

        @pl.when(ch < NRC)
        def _():
            pltpu.sync_copy(rowbuf, acc.at[pl.ds(ch * RCH, RCH)])

        return 0

    lax.fori_loop(0, RITER, zero_body, 0)
    plsc.subcore_barrier()

    for which, norm_hbm in ((0, nsh_hbm), (1, ndh_hbm)):
        def deg_body(i, _):
            off = s * EPS + i * CH
            if which == 0:
                pltpu.sync_copy(src2_hbm.at[pl.ds(off, CH)], idxs)
            else:
                pltpu.sync_copy(dst_hbm.at[pl.ds(off, CH)], idxs)
            pltpu.sync_copy(msg, acc.at[idxs], add=True)
            return 0

        lax.fori_loop(0, NCH, deg_body, 0)
        plsc.subcore_barrier()

        def extract_body(i, _):
            ch = i * NS + s

            @pl.when(ch < NRC)
            def _():
                row0 = ch * RCH
                pltpu.sync_copy(acc.at[pl.ds(row0, RCH)], rowbuf)

                def nrow(r, _c):
                    nbuf[r, :] = _rsqrt16(
                        jnp.maximum(rowbuf[r, pl.ds(0, LANES)], 1.0))
                    return 0

                lax.fori_loop(0, RCH, nrow, 0)
                pltpu.sync_copy(nbuf, norm_hbm.at[pl.ds(row0, RCH)])
                lax.fori_loop(0, RCH, zero_rowbuf, 0)
                pltpu.sync_copy(rowbuf, acc.at[pl.ds(row0, RCH)])

            return 0

        lax.fori_loop(0, RITER, extract_body, 0)
        plsc.subcore_barrier()

    def init_body(i, _):
        ch = i * NS + s

        @pl.when(ch < NRC)
        def _():
            row0 = ch * RCH
            grow0 = c * N + row0
            pltpu.sync_copy(nsh_hbm.at[pl.ds(row0, RCH)], nbuf)
            pltpu.sync_copy(x0_hbm.at[pl.ds(grow0, RCH)], rowbuf)
            lax.fori_loop(0, RCH, srow, 0)
            pltpu.sync_copy(rowbuf, xs_hbm.at[pl.ds(grow0, RCH)])

        return 0

    lax.fori_loop(0, RITER, init_body, 0)
    plsc.subcore_barrier()

    for k in range(ORDER):
        last = k == ORDER - 1

        ebase = s * EPS
        pltpu.sync_copy(src2_hbm.at[pl.ds(c * E + ebase, CH)], idxs)
        pltpu.sync_copy(dst_hbm.at[pl.ds(ebase, CH)], idxd)
        g0 = pltpu.async_copy(xs_hbm.at[idxs], msg, sem)

        def pair_body(p, _):
            off_b = ebase + (2 * p + 1) * CH
            pltpu.sync_copy(src2_hbm.at[pl.ds(c * E + off_b, CH)], idxs2)
            pltpu.sync_copy(dst_hbm.at[pl.ds(off_b, CH)], idxd2)
            gb = pltpu.async_copy(xs_hbm.at[idxs2], msg2, sem2)
            pltpu.make_async_copy(xs_hbm.at[idxs], msg, sem).wait()
            pltpu.sync_copy(msg, acc.at[idxd], add=True)

            @pl.when(p + 1 < NCH // 2)
            def _():
                off_a = ebase + (2 * p + 2) * CH
                pltpu.sync_copy(src2_hbm.at[pl.ds(c * E + off_a, CH)], idxs)
                pltpu.sync_copy(dst_hbm.at[pl.ds(off_a, CH)], idxd)
                pltpu.async_copy(xs_hbm.at[idxs], msg, sem)

            pltpu.make_async_copy(xs_hbm.at[idxs2], msg2, sem2).wait()
            pltpu.sync_copy(msg2, acc.at[idxd2], add=True)
            return 0

        lax.fori_loop(0, NCH // 2, pair_body, 0)
        plsc.subcore_barrier()

        def epi_body(i, _):
            ch = i * NS + s

            @pl.when(ch < NRC)
            def _():
                row0 = ch * RCH
                grow0 = c * N + row0
                pltpu.sync_copy(acc.at[pl.ds(row0, RCH)], rowbuf)
                pltpu.sync_copy(ndh_hbm.at[pl.ds(row0, RCH)], nbuf)
                lax.fori_loop(0, RCH, srow, 0)
                pltpu.sync_copy(rowbuf, nx_hbm[k].at[pl.ds(grow0, RCH)])
                if not last:
                    pltpu.sync_copy(nsh_hbm.at[pl.ds(row0, RCH)], nbuf)
                    lax.fori_loop(0, RCH, srow, 0)
                    pltpu.sync_copy(rowbuf, xs_hbm.at[pl.ds(grow0, RCH)])
                    lax.fori_loop(0, RCH, zero_rowbuf, 0)
                    pltpu.sync_copy(rowbuf, acc.at[pl.ds(row0, RCH)])

            return 0

        lax.fori_loop(0, RITER, epi_body, 0)
        if not last:
            plsc.subcore_barrier()


_propagate = pl.kernel(
    _prop_body,
    out_type=[jax.ShapeDtypeStruct((SAMPLE * N, D), jnp.float32)
              for _ in range(ORDER + 1)]
             + [jax.ShapeDtypeStruct((N, LANES), jnp.float32)
                for _ in range(2)],
    mesh=plsc.VectorSubcoreMesh(core_axis_name="c", subcore_axis_name="s"),
    scratch_types=[
        pltpu.VMEM_SHARED((N, D), jnp.float32),
        pltpu.VMEM((CH,), jnp.int32),
        pltpu.VMEM((CH,), jnp.int32),
        pltpu.VMEM((CH,), jnp.int32),
        pltpu.VMEM((CH,), jnp.int32),
        pltpu.VMEM((CH, D), jnp.float32),
        pltpu.VMEM((CH, D), jnp.float32),
        pltpu.VMEM((RCH, D), jnp.float32),
        pltpu.VMEM((RCH, LANES), jnp.float32),
        pltpu.SemaphoreType.DMA,
        pltpu.SemaphoreType.DMA,
    ],
)


def _mlp_body(x0_ref, x1_ref, x2_ref, x3_ref, x4_ref,
              w1_ref, b1_ref, w2_ref, b2_ref, out_ref):
    y = (x0_ref[...] + x1_ref[...] + x2_ref[...] + x3_ref[...] + x4_ref[...])
    y = y * (1.0 / (ORDER + 1))
    h = jnp.dot(y, w1_ref[...], preferred_element_type=jnp.float32) + b1_ref[...]
    h = jnp.maximum(h, 0.0)
    o = jnp.dot(h, w2_ref[...], preferred_element_type=jnp.float32) + b2_ref[...]
    m = jnp.max(o, axis=-1, keepdims=True)
    lse = jnp.log(jnp.sum(jnp.exp(o - m), axis=-1, keepdims=True)) + m
    out_ref[...] = o - lse


_MLP_ROWS = 2000

_mlp = pl.pallas_call(
    _mlp_body,
    grid=(SAMPLE * N // _MLP_ROWS,),
    in_specs=[
        pl.BlockSpec((_MLP_ROWS, D), lambda i: (i, 0)),
        pl.BlockSpec((_MLP_ROWS, D), lambda i: (i, 0)),
        pl.BlockSpec((_MLP_ROWS, D), lambda i: (i, 0)),
        pl.BlockSpec((_MLP_ROWS, D), lambda i: (i, 0)),
        pl.BlockSpec((_MLP_ROWS, D), lambda i: (i, 0)),
        pl.BlockSpec((D, H), lambda i: (0, 0)),
        pl.BlockSpec((1, H), lambda i: (0, 0)),
        pl.BlockSpec((H, C), lambda i: (0, 0)),
        pl.BlockSpec((1, C), lambda i: (0, 0)),
    ],
    out_specs=pl.BlockSpec((_MLP_ROWS, C), lambda i: (i, 0)),
    out_shape=jax.ShapeDtypeStruct((SAMPLE * N, C), jnp.float32),
)


@jax.jit
def kernel(feats, edge_index, W1, b1, W2, b2):
    src = edge_index[0]
    dst = edge_index[1]

    masks = []
    for sm in range(SAMPLE):
        mk = jax.random.fold_in(jax.random.key(1), sm)
        masks.append(
            jax.random.bernoulli(mk, 1.0 - P_DROP_NODE, (N, 1)).astype(jnp.float32)
        )
    x0 = jnp.concatenate([feats * m for m in masks], axis=0)

    src2 = jnp.concatenate([src, src + N], axis=0)

    _, nx1, nx2, nx3, nx4, _, _ = _propagate(x0, src2, dst)
    logits = _mlp(x0, nx1, nx2, nx3, nx4,
                  W1, b1.reshape(1, H), W2, b2.reshape(1, C))
    return logits.reshape(SAMPLE, N, C)


# speedup vs baseline: 6.3216x; 1.1099x over previous
import jax
import jax.numpy as jnp
from jax import lax
from jax.experimental import pallas as pl
from jax.experimental.pallas import tpu as pltpu, tpu_sc as plsc

N = 10000
E = 320000
D = 128
H = 256
C = 64
SAMPLE = 2
ORDER = 4
P_DROP_NODE = 0.5

NS = 16
LANES = 16
EPS = E // NS
CH = 80
NCH = EPS // CH
RCH = 80
NRC = N // RCH
RITER = (NRC + NS - 1) // NS
DV = D // LANES


def _rsqrt16(x):
    y = jnp.full((LANES,), 1.0, jnp.float32)
    for k in range(1, 19):
        y = jnp.where(x >= float(1 << k), jnp.float32(2.0 ** (-k / 2.0)), y)
    for _ in range(5):
        y = y * (1.5 - 0.5 * x * y * y)
    return y


def _prop_body(x0_hbm, src2_hbm, dst_hbm,
               xs_hbm, nx1_hbm, nx2_hbm, nx3_hbm, nx4_hbm, nsh_hbm, ndh_hbm,
               acc, idxs, idxd, idxs2, idxd2, msg, msg2, rowbuf, nbuf, sem, sem2):
    c = lax.axis_index("c")
    s = lax.axis_index("s")
    nx_hbm = [nx1_hbm, nx2_hbm, nx3_hbm, nx4_hbm]

    def zero_rowbuf(r, _):
        for j in range(DV):
            rowbuf[r, pl.ds(LANES * j, LANES)] = jnp.zeros((LANES,), jnp.float32)
        return 0

    def srow(r, _c):
        nv = nbuf[r, :]
        for j in range(DV):
            sl = pl.ds(LANES * j, LANES)
            rowbuf[r, sl] = rowbuf[r, sl] * nv
        return 0

    def fill_body(r, _):
        zero_rowbuf(r, 0)
        for j in range(DV):
            msg[r, pl.ds(LANES * j, LANES)] = jnp.ones((LANES,), jnp.float32)
            msg2[r, pl.ds(LANES * j, LANES)] = jnp.ones((LANES,), jnp.float32)
        return 0

    lax.fori_loop(0, RCH, fill_body, 0)

    def zero_body(i, _):
        ch = i * NS + s

        @pl.when(ch < NRC)
        def _():
            pltpu.sync_copy(rowbuf, acc.at[pl.ds(ch * RCH, RCH)])

        return 0

    lax.fori_loop(0, RITER, zero_body, 0)
    plsc.subcore_barrier()

    for which, norm_hbm in ((0, nsh_hbm), (1, ndh_hbm)):
        eidx_hbm = src2_hbm if which == 0 else dst_hbm

        def dload(buf, chunk):
            pltpu.sync_copy(eidx_hbm.at[pl.ds(s * EPS + chunk * CH, CH)], buf)

        dload(idxs, 0)

        def dpair(p, _):
            a1 = pltpu.async_copy(msg, acc.at[idxs], add=True, sem=sem)
            dload(idxs2, 2 * p + 1)
            a2 = pltpu.async_copy(msg2, acc.at[idxs2], add=True, sem=sem2)
            pltpu.make_async_copy(msg, acc.at[idxs], sem).wait()

            @pl.when(p + 1 < NCH // 2)
            def _():
                dload(idxs, 2 * p + 2)

            pltpu.make_async_copy(msg2, acc.at[idxs2], sem2).wait()
            return 0

        lax.fori_loop(0, NCH // 2, dpair, 0)
        plsc.subcore_barrier()

        def extract_body(i, _):
            ch = i * NS + s

            @pl.when(ch < NRC)
            def _():
                row0 = ch * RCH
                pltpu.sync_copy(acc.at[pl.ds(row0, RCH)], rowbuf)

                def nrow(r, _c):
                    nbuf[r, :] = _rsqrt16(
                        jnp.maximum(rowbuf[r, pl.ds(0, LANES)], 1.0))
                    return 0

                lax.fori_loop(0, RCH, nrow, 0)
                pltpu.sync_copy(nbuf, norm_hbm.at[pl.ds(row0, RCH)])
                lax.fori_loop(0, RCH, zero_rowbuf, 0)
                pltpu.sync_copy(rowbuf, acc.at[pl.ds(row0, RCH)])

            return 0

        lax.fori_loop(0, RITER, extract_body, 0)
        plsc.subcore_barrier()

    def init_body(i, _):
        ch = i * NS + s

        @pl.when(ch < NRC)
        def _():
            row0 = ch * RCH
            grow0 = c * N + row0
            pltpu.sync_copy(nsh_hbm.at[pl.ds(row0, RCH)], nbuf)
            pltpu.sync_copy(x0_hbm.at[pl.ds(grow0, RCH)], rowbuf)
            lax.fori_loop(0, RCH, srow, 0)
            pltpu.sync_copy(rowbuf, xs_hbm.at[pl.ds(grow0, RCH)])

        return 0

    lax.fori_loop(0, RITER, init_body, 0)
    plsc.subcore_barrier()

    for k in range(ORDER):
        last = k == ORDER - 1

        ebase = s * EPS
        pltpu.sync_copy(src2_hbm.at[pl.ds(c * E + ebase, CH)], idxs)
        pltpu.sync_copy(dst_hbm.at[pl.ds(ebase, CH)], idxd)
        g0 = pltpu.async_copy(xs_hbm.at[idxs], msg, sem)

        def pair_body(p, _):
            off_b = ebase + (2 * p + 1) * CH
            pltpu.sync_copy(src2_hbm.at[pl.ds(c * E + off_b, CH)], idxs2)
            pltpu.sync_copy(dst_hbm.at[pl.ds(off_b, CH)], idxd2)
            gb = pltpu.async_copy(xs_hbm.at[idxs2], msg2, sem2)
            pltpu.make_async_copy(xs_hbm.at[idxs], msg, sem).wait()
            pltpu.sync_copy(msg, acc.at[idxd], add=True)

            @pl.when(p + 1 < NCH // 2)
            def _():
                off_a = ebase + (2 * p + 2) * CH
                pltpu.sync_copy(src2_hbm.at[pl.ds(c * E + off_a, CH)], idxs)
                pltpu.sync_copy(dst_hbm.at[pl.ds(off_a, CH)], idxd)
                pltpu.async_copy(xs_hbm.at[idxs], msg, sem)

            pltpu.make_async_copy(xs_hbm.at[idxs2], msg2, sem2).wait()
            pltpu.sync_copy(msg2, acc.at[idxd2], add=True)
            return 0

        lax.fori_loop(0, NCH // 2, pair_body, 0)
        plsc.subcore_barrier()

        def epi_body(i, _):
            ch = i * NS + s

            @pl.when(ch < NRC)
            def _():
                row0 = ch * RCH
                grow0 = c * N + row0
                pltpu.sync_copy(acc.at[pl.ds(row0, RCH)], rowbuf)
                pltpu.sync_copy(ndh_hbm.at[pl.ds(row0, RCH)], nbuf)
                lax.fori_loop(0, RCH, srow, 0)
                pltpu.sync_copy(rowbuf, nx_hbm[k].at[pl.ds(grow0, RCH)])
                if not last:
                    pltpu.sync_copy(nsh_hbm.at[pl.ds(row0, RCH)], nbuf)
                    lax.fori_loop(0, RCH, srow, 0)
                    pltpu.sync_copy(rowbuf, xs_hbm.at[pl.ds(grow0, RCH)])
                    lax.fori_loop(0, RCH, zero_rowbuf, 0)
                    pltpu.sync_copy(rowbuf, acc.at[pl.ds(row0, RCH)])

            return 0

        lax.fori_loop(0, RITER, epi_body, 0)
        if not last:
            plsc.subcore_barrier()


_propagate = pl.kernel(
    _prop_body,
    out_type=[jax.ShapeDtypeStruct((SAMPLE * N, D), jnp.float32)
              for _ in range(ORDER + 1)]
             + [jax.ShapeDtypeStruct((N, LANES), jnp.float32)
                for _ in range(2)],
    mesh=plsc.VectorSubcoreMesh(core_axis_name="c", subcore_axis_name="s"),
    scratch_types=[
        pltpu.VMEM_SHARED((N, D), jnp.float32),
        pltpu.VMEM((CH,), jnp.int32),
        pltpu.VMEM((CH,), jnp.int32),
        pltpu.VMEM((CH,), jnp.int32),
        pltpu.VMEM((CH,), jnp.int32),
        pltpu.VMEM((CH, D), jnp.float32),
        pltpu.VMEM((CH, D), jnp.float32),
        pltpu.VMEM((RCH, D), jnp.float32),
        pltpu.VMEM((RCH, LANES), jnp.float32),
        pltpu.SemaphoreType.DMA,
        pltpu.SemaphoreType.DMA,
    ],
)


def _mlp_body(x0_ref, x1_ref, x2_ref, x3_ref, x4_ref,
              w1_ref, b1_ref, w2_ref, b2_ref, out_ref):
    y = (x0_ref[...] + x1_ref[...] + x2_ref[...] + x3_ref[...] + x4_ref[...])
    y = y * (1.0 / (ORDER + 1))
    h = jnp.dot(y, w1_ref[...], preferred_element_type=jnp.float32) + b1_ref[...]
    h = jnp.maximum(h, 0.0)
    o = jnp.dot(h, w2_ref[...], preferred_element_type=jnp.float32) + b2_ref[...]
    m = jnp.max(o, axis=-1, keepdims=True)
    lse = jnp.log(jnp.sum(jnp.exp(o - m), axis=-1, keepdims=True)) + m
    out_ref[...] = o - lse


_MLP_ROWS = 2000

_mlp = pl.pallas_call(
    _mlp_body,
    grid=(SAMPLE * N // _MLP_ROWS,),
    in_specs=[
        pl.BlockSpec((_MLP_ROWS, D), lambda i: (i, 0)),
        pl.BlockSpec((_MLP_ROWS, D), lambda i: (i, 0)),
        pl.BlockSpec((_MLP_ROWS, D), lambda i: (i, 0)),
        pl.BlockSpec((_MLP_ROWS, D), lambda i: (i, 0)),
        pl.BlockSpec((_MLP_ROWS, D), lambda i: (i, 0)),
        pl.BlockSpec((D, H), lambda i: (0, 0)),
        pl.BlockSpec((1, H), lambda i: (0, 0)),
        pl.BlockSpec((H, C), lambda i: (0, 0)),
        pl.BlockSpec((1, C), lambda i: (0, 0)),
    ],
    out_specs=pl.BlockSpec((_MLP_ROWS, C), lambda i: (i, 0)),
    out_shape=jax.ShapeDtypeStruct((SAMPLE * N, C), jnp.float32),
)


@jax.jit
def kernel(feats, edge_index, W1, b1, W2, b2):
    src = edge_index[0]
    dst = edge_index[1]

    masks = []
    for sm in range(SAMPLE):
        mk = jax.random.fold_in(jax.random.key(1), sm)
        masks.append(
            jax.random.bernoulli(mk, 1.0 - P_DROP_NODE, (N, 1)).astype(jnp.float32)
        )
    x0 = jnp.concatenate([feats * m for m in masks], axis=0)

    src2 = jnp.concatenate([src, src + N], axis=0)

    _, nx1, nx2, nx3, nx4, _, _ = _propagate(x0, src2, dst)
    logits = _mlp(x0, nx1, nx2, nx3, nx4,
                  W1, b1.reshape(1, H), W2, b2.reshape(1, C))
    return logits.reshape(SAMPLE, N, C)
